# baseline (speedup 1.0000x reference)
import functools

import jax
import jax.numpy as jnp
from jax import lax
from jax.experimental import pallas as pl
from jax.experimental.pallas import tpu as pltpu
from jax.experimental.pallas import tpu_sc as plsc

N_CLASSES = 8192
EMBED = 64
BATCH = 16384
X_MAX = 100.0
ALPHA = 0.75

NC, NS, L = 2, 16, 16
NW = NC * NS
BPW = BATCH // NW
CHUNK = 128
NCHUNK = BPW // CHUNK
NPASS = BPW // CHUNK
GPP = CHUNK // L
NGRP = BPW // L

LN2 = 0.6931471805599453
LN_XMAX = 4.605170185988091
SQRT2 = 1.4142135623730951


def _vln(x):
    bits = lax.bitcast_convert_type(x, jnp.int32)
    e = lax.shift_right_arithmetic(bits, 23) - 127
    m = lax.bitcast_convert_type(
        lax.bitwise_or(lax.bitwise_and(bits, 0x007FFFFF), 0x3F800000),
        jnp.float32)
    big = m > SQRT2
    m = jnp.where(big, m * 0.5, m)
    e = e + jnp.where(big, 1, 0)
    t = (m - 1.0) / (m + 1.0)
    t2 = t * t
    p = t * (2.0 + t2 * (2.0 / 3.0 + t2 * (0.4 + t2 * (2.0 / 7.0))))
    return e.astype(jnp.float32) * LN2 + p


_MESH = plsc.VectorSubcoreMesh(core_axis_name="c", subcore_axis_name="s")


@functools.partial(
    pl.kernel,
    out_type=jax.ShapeDtypeStruct((NW, L), jnp.float32),
    mesh=_MESH,
    compiler_params=pltpu.CompilerParams(needs_layout_passes=False),
    scratch_types=[
        pltpu.VMEM((BPW,), jnp.int32),
        pltpu.VMEM((BPW,), jnp.int32),
        pltpu.VMEM((BPW,), jnp.int32),
        pltpu.VMEM((BPW,), jnp.int32),
        pltpu.VMEM((BPW,), jnp.int32),
        pltpu.VMEM((BPW,), jnp.int32),
        pltpu.VMEM((BPW,), jnp.float32),
        pltpu.VMEM((BPW,), jnp.float32),
        pltpu.VMEM((BPW,), jnp.float32),
        pltpu.VMEM((BPW,), jnp.float32),
        pltpu.VMEM((2, CHUNK, 2 * EMBED), jnp.float32),
        pltpu.VMEM((2, CHUNK, 2 * EMBED), jnp.float32),
        pltpu.VMEM((L,), jnp.float32),
        pltpu.SemaphoreType.DMA,
        pltpu.SemaphoreType.DMA,
    ],
)
def _glove_sc(inp_hbm, outp_hbm, co_hbm, win_hbm, wout_hbm, bb_hbm, out_hbm,
              inp_v, outp_v, rin_v, rout_v, lin_v, ob_v, co_v, bin_v, bout_v,
              pred_v, win_b, wout_b, part_v, sem, rsem):
    wid = lax.axis_index("s") * NC + lax.axis_index("c")
    base = wid * BPW

    with jax.named_scope("p_stage_idx"):
        pltpu.sync_copy(inp_hbm.at[pl.ds(base, BPW)], inp_v)
        pltpu.sync_copy(outp_hbm.at[pl.ds(base, BPW)], outp_v)

    for k in range(NGRP):
        sl = pl.ds(k * L, L)
        rin_v[sl] = lax.shift_right_logical(inp_v[sl], 1)
        rout_v[sl] = lax.shift_right_logical(outp_v[sl], 1)

    def fire(t):
        sl = pl.ds(t * CHUNK, CHUNK)
        return (pltpu.async_copy(win_hbm.at[rin_v.at[sl]], win_b.at[t % 2], rsem),
                pltpu.async_copy(wout_hbm.at[rout_v.at[sl]], wout_b.at[t % 2], rsem))

    pend = fire(0)

    for k in range(NGRP):
        sl = pl.ds(k * L, L)
        a = inp_v[sl]
        b = outp_v[sl]
        lin_v[sl] = (lax.shift_left(lax.shift_right_logical(a, 3), 16) |
                     lax.shift_left(lax.shift_right_logical(b, 7), 10) |
                     lax.shift_left(lax.bitwise_and(a, 7), 7) |
                     lax.bitwise_and(b, 127))
        ob_v[sl] = b + N_CLASSES

    small = []
    for j in range(NCHUNK):
        sl = pl.ds(j * CHUNK, CHUNK)
        small.append(pltpu.async_copy(co_hbm.at[lin_v.at[sl]], co_v.at[sl], sem))
        small.append(pltpu.async_copy(bb_hbm.at[inp_v.at[sl]], bin_v.at[sl], sem))
        small.append(pltpu.async_copy(bb_hbm.at[ob_v.at[sl]], bout_v.at[sl], sem))

    for t in range(NPASS):
        nxt = fire(t + 1) if t + 1 < NPASS else None
        with jax.named_scope(f"p_row_wait{t}"):
            pend[0].wait()
            pend[1].wait()
        pend = nxt
        wbuf = win_b.at[t % 2]
        obuf = wout_b.at[t % 2]

        def dots(gl, _, wbuf=wbuf, obuf=obuf, t=t):
            sl = pl.ds(t * CHUNK + gl * L, L)
            ii = lax.iota(jnp.int32, L)
            rows = ii + gl * L
            cin = lax.bitwise_and(inp_v[sl], 1) * EMBED
            cout = lax.bitwise_and(outp_v[sl], 1) * EMBED
            acc4 = [jnp.zeros((L,), jnp.float32) for _ in range(4)]
            for d in range(EMBED):
                dd = lax.bitwise_and(ii + d, EMBED - 1)
                acc4[d % 4] = acc4[d % 4] + (
                    plsc.load_gather(wbuf, [rows, cin + dd]) *
                    plsc.load_gather(obuf, [rows, cout + dd]))
            pred_v[sl] = (acc4[0] + acc4[1]) + (acc4[2] + acc4[3])
            return 0

        with jax.named_scope(f"p_comp{t}"):
            lax.fori_loop(0, GPP, dots, 0, unroll=4)

    with jax.named_scope("p_co_wait"):
        for c in small:
            c.wait()

    def loss(g, acc):
        sl = pl.ds(g * L, L)
        co = co_v[sl] + 1.0
        lnco = _vln(co)
        w = jnp.where(co > X_MAX, 1.0, jnp.exp(ALPHA * (lnco - LN_XMAX)))
        diff = pred_v[sl] + bin_v[sl] + bout_v[sl] - lnco
        return acc + diff * diff * w

    with jax.named_scope("p_loss"):
        part_v[...] = lax.fori_loop(0, NGRP, loss, jnp.zeros((L,), jnp.float32))
    pltpu.sync_copy(part_v, out_hbm.at[wid])


def kernel(input, output, co_oc, W_in, b_in, W_out, b_out):
    co_phys = co_oc.reshape(1024, 8, 64, 128).transpose(0, 2, 1, 3).reshape(-1)
    bb = jnp.concatenate([b_in.reshape(-1), b_out.reshape(-1)])
    parts = _glove_sc(input, output, co_phys,
                      W_in.reshape(N_CLASSES // 2, 2 * EMBED),
                      W_out.reshape(N_CLASSES // 2, 2 * EMBED), bb)
    return jnp.sum(parts)


# speedup vs baseline: 1.0590x; 1.0590x over previous
import functools

import jax
import jax.numpy as jnp
from jax import lax
from jax.experimental import pallas as pl
from jax.experimental.pallas import tpu as pltpu
from jax.experimental.pallas import tpu_sc as plsc

N_CLASSES = 8192
EMBED = 64
BATCH = 16384
X_MAX = 100.0
ALPHA = 0.75

NC, NS, L = 2, 16, 16
NW = NC * NS
BPW = BATCH // NW
CHUNK = 128
NCHUNK = BPW // CHUNK
NPASS = BPW // CHUNK
GPP = CHUNK // L
NGRP = BPW // L

LN2 = 0.6931471805599453
LN_XMAX = 4.605170185988091
SQRT2 = 1.4142135623730951


def _vln(x):
    bits = lax.bitcast_convert_type(x, jnp.int32)
    e = lax.shift_right_arithmetic(bits, 23) - 127
    m = lax.bitcast_convert_type(
        lax.bitwise_or(lax.bitwise_and(bits, 0x007FFFFF), 0x3F800000),
        jnp.float32)
    big = m > SQRT2
    m = jnp.where(big, m * 0.5, m)
    e = e + jnp.where(big, 1, 0)
    t = (m - 1.0) / (m + 1.0)
    t2 = t * t
    p = t * (2.0 + t2 * (2.0 / 3.0 + t2 * (0.4 + t2 * (2.0 / 7.0))))
    return e.astype(jnp.float32) * LN2 + p


_MESH = plsc.VectorSubcoreMesh(core_axis_name="c", subcore_axis_name="s")


@functools.partial(
    pl.kernel,
    out_type=jax.ShapeDtypeStruct((NW, L), jnp.float32),
    mesh=_MESH,
    compiler_params=pltpu.CompilerParams(needs_layout_passes=False),
    scratch_types=[
        pltpu.VMEM((BPW,), jnp.int32),
        pltpu.VMEM((BPW,), jnp.int32),
        pltpu.VMEM((BPW,), jnp.int32),
        pltpu.VMEM((BPW,), jnp.int32),
        pltpu.VMEM((BPW,), jnp.int32),
        pltpu.VMEM((BPW,), jnp.int32),
        pltpu.VMEM((BPW,), jnp.float32),
        pltpu.VMEM((BPW,), jnp.float32),
        pltpu.VMEM((BPW,), jnp.float32),
        pltpu.VMEM((BPW,), jnp.float32),
        pltpu.VMEM((2, CHUNK, 2 * EMBED), jnp.float32),
        pltpu.VMEM((2, CHUNK, 2 * EMBED), jnp.float32),
        pltpu.VMEM((L,), jnp.float32),
        pltpu.SemaphoreType.DMA,
        pltpu.SemaphoreType.DMA,
    ],
)
def _glove_sc(inp_hbm, outp_hbm, co_hbm, win_hbm, wout_hbm, bb_hbm, out_hbm,
              inp_v, outp_v, rin_v, rout_v, lin_v, ob_v, co_v, bin_v, bout_v,
              pred_v, win_b, wout_b, part_v, sem, rsem):
    wid = lax.axis_index("s") * NC + lax.axis_index("c")
    base = wid * BPW

    with jax.named_scope("p_stage_idx"):
        c1 = pltpu.async_copy(inp_hbm.at[pl.ds(base, BPW)], inp_v, sem)
        c2 = pltpu.async_copy(outp_hbm.at[pl.ds(base, BPW)], outp_v, sem)
        c1.wait()
        c2.wait()

    for k in range(NGRP):
        sl = pl.ds(k * L, L)
        rin_v[sl] = lax.shift_right_logical(inp_v[sl], 1)
        rout_v[sl] = lax.shift_right_logical(outp_v[sl], 1)

    HALF = CHUNK // 2

    def fire(t):
        sl = pl.ds(t * CHUNK, CHUNK)
        return (pltpu.async_copy(win_hbm.at[rin_v.at[sl]], win_b.at[t % 2], rsem),
                pltpu.async_copy(wout_hbm.at[rout_v.at[sl]], wout_b.at[t % 2], rsem))

    pend0a = (pltpu.async_copy(win_hbm.at[rin_v.at[pl.ds(0, HALF)]],
                               win_b.at[0, pl.ds(0, HALF)], rsem),
              pltpu.async_copy(wout_hbm.at[rout_v.at[pl.ds(0, HALF)]],
                               wout_b.at[0, pl.ds(0, HALF)], rsem))
    pend0b = (pltpu.async_copy(win_hbm.at[rin_v.at[pl.ds(HALF, HALF)]],
                               win_b.at[0, pl.ds(HALF, HALF)], rsem),
              pltpu.async_copy(wout_hbm.at[rout_v.at[pl.ds(HALF, HALF)]],
                               wout_b.at[0, pl.ds(HALF, HALF)], rsem))
    pend = None

    for k in range(NGRP):
        sl = pl.ds(k * L, L)
        a = inp_v[sl]
        b = outp_v[sl]
        lin_v[sl] = (lax.shift_left(lax.shift_right_logical(a, 3), 16) |
                     lax.shift_left(lax.shift_right_logical(b, 7), 10) |
                     lax.shift_left(lax.bitwise_and(a, 7), 7) |
                     lax.bitwise_and(b, 127))
        ob_v[sl] = b + N_CLASSES

    small = []
    for j in range(NCHUNK):
        sl = pl.ds(j * CHUNK, CHUNK)
        small.append(pltpu.async_copy(co_hbm.at[lin_v.at[sl]], co_v.at[sl], sem))
        small.append(pltpu.async_copy(bb_hbm.at[inp_v.at[sl]], bin_v.at[sl], sem))
        small.append(pltpu.async_copy(bb_hbm.at[ob_v.at[sl]], bout_v.at[sl], sem))

    for t in range(NPASS):
        nxt = fire(t + 1) if t + 1 < NPASS else None
        if t == 0:
            with jax.named_scope("p_row_wait0a"):
                pend0a[0].wait()
                pend0a[1].wait()
        else:
            with jax.named_scope(f"p_row_wait{t}"):
                pend[0].wait()
                pend[1].wait()
        pend = nxt
        wbuf = win_b.at[t % 2]
        obuf = wout_b.at[t % 2]

        def dots(gl, _, wbuf=wbuf, obuf=obuf, t=t):
            sl = pl.ds(t * CHUNK + gl * L, L)
            ii = lax.iota(jnp.int32, L)
            rows = ii + gl * L
            cin = lax.bitwise_and(inp_v[sl], 1) * EMBED
            cout = lax.bitwise_and(outp_v[sl], 1) * EMBED
            acc4 = [jnp.zeros((L,), jnp.float32) for _ in range(4)]
            for d in range(EMBED):
                dd = lax.bitwise_and(ii + d, EMBED - 1)
                acc4[d % 4] = acc4[d % 4] + (
                    plsc.load_gather(wbuf, [rows, cin + dd]) *
                    plsc.load_gather(obuf, [rows, cout + dd]))
            pred_v[sl] = (acc4[0] + acc4[1]) + (acc4[2] + acc4[3])
            return 0

        if t == 0:
            with jax.named_scope("p_comp0a"):
                lax.fori_loop(0, GPP // 2, dots, 0, unroll=2)
            with jax.named_scope("p_row_wait0b"):
                pend0b[0].wait()
                pend0b[1].wait()
            with jax.named_scope("p_comp0b"):
                lax.fori_loop(GPP // 2, GPP, dots, 0, unroll=2)
        else:
            with jax.named_scope(f"p_comp{t}"):
                lax.fori_loop(0, GPP, dots, 0, unroll=2)

    with jax.named_scope("p_co_wait"):
        for c in small:
            c.wait()

    def loss(g, acc):
        sl = pl.ds(g * L, L)
        co = co_v[sl] + 1.0
        lnco = _vln(co)
        w = jnp.where(co > X_MAX, 1.0, jnp.exp(ALPHA * (lnco - LN_XMAX)))
        diff = pred_v[sl] + bin_v[sl] + bout_v[sl] - lnco
        return acc + diff * diff * w

    with jax.named_scope("p_loss"):
        part_v[...] = lax.fori_loop(0, NGRP, loss, jnp.zeros((L,), jnp.float32))
    pltpu.sync_copy(part_v, out_hbm.at[wid])


def kernel(input, output, co_oc, W_in, b_in, W_out, b_out):
    co_phys = co_oc.reshape(1024, 8, 64, 128).transpose(0, 2, 1, 3).reshape(-1)
    bb = jnp.concatenate([b_in.reshape(-1), b_out.reshape(-1)])
    parts = _glove_sc(input, output, co_phys,
                      W_in.reshape(N_CLASSES // 2, 2 * EMBED),
                      W_out.reshape(N_CLASSES // 2, 2 * EMBED), bb)
    return jnp.sum(parts)


# speedup vs baseline: 1.0786x; 1.0185x over previous
import functools

import jax
import jax.numpy as jnp
from jax import lax
from jax.experimental import pallas as pl
from jax.experimental.pallas import tpu as pltpu
from jax.experimental.pallas import tpu_sc as plsc

N_CLASSES = 8192
EMBED = 64
BATCH = 16384
X_MAX = 100.0
ALPHA = 0.75

NC, NS, L = 2, 16, 16
NW = NC * NS
BPW = BATCH // NW
CHUNK = 128
NCHUNK = BPW // CHUNK
NPASS = BPW // CHUNK
GPP = CHUNK // L
NGRP = BPW // L

LN2 = 0.6931471805599453
LN_XMAX = 4.605170185988091
SQRT2 = 1.4142135623730951


def _vln(x):
    bits = lax.bitcast_convert_type(x, jnp.int32)
    e = lax.shift_right_arithmetic(bits, 23) - 127
    m = lax.bitcast_convert_type(
        lax.bitwise_or(lax.bitwise_and(bits, 0x007FFFFF), 0x3F800000),
        jnp.float32)
    big = m > SQRT2
    m = jnp.where(big, m * 0.5, m)
    e = e + jnp.where(big, 1, 0)
    t = (m - 1.0) / (m + 1.0)
    t2 = t * t
    p = t * (2.0 + t2 * (2.0 / 3.0 + t2 * (0.4 + t2 * (2.0 / 7.0))))
    return e.astype(jnp.float32) * LN2 + p


_MESH = plsc.VectorSubcoreMesh(core_axis_name="c", subcore_axis_name="s")


@functools.partial(
    pl.kernel,
    out_type=jax.ShapeDtypeStruct((NW, L), jnp.float32),
    mesh=_MESH,
    compiler_params=pltpu.CompilerParams(needs_layout_passes=False),
    scratch_types=[
        pltpu.VMEM((BPW,), jnp.int32),
        pltpu.VMEM((BPW,), jnp.int32),
        pltpu.VMEM((BPW,), jnp.int32),
        pltpu.VMEM((BPW,), jnp.int32),
        pltpu.VMEM((BPW,), jnp.int32),
        pltpu.VMEM((BPW,), jnp.int32),
        pltpu.VMEM((BPW,), jnp.float32),
        pltpu.VMEM((BPW,), jnp.float32),
        pltpu.VMEM((BPW,), jnp.float32),
        pltpu.VMEM((BPW,), jnp.float32),
        pltpu.VMEM((2, CHUNK, 2 * EMBED), jnp.float32),
        pltpu.VMEM((2, CHUNK, 2 * EMBED), jnp.float32),
        pltpu.VMEM((L,), jnp.float32),
        pltpu.SemaphoreType.DMA,
        pltpu.SemaphoreType.DMA,
    ],
)
def _glove_sc(inp_hbm, outp_hbm, co_hbm, win_hbm, wout_hbm, bb_hbm, out_hbm,
              inp_v, outp_v, rin_v, rout_v, lin_v, ob_v, co_v, bin_v, bout_v,
              pred_v, win_b, wout_b, part_v, sem, rsem):
    wid = lax.axis_index("s") * NC + lax.axis_index("c")
    base = wid * BPW

    pltpu.sync_copy(inp_hbm.at[pl.ds(base, BPW)], inp_v)
    pltpu.sync_copy(outp_hbm.at[pl.ds(base, BPW)], outp_v)

    for k in range(NGRP):
        sl = pl.ds(k * L, L)
        rin_v[sl] = lax.shift_right_logical(inp_v[sl], 1)
        rout_v[sl] = lax.shift_right_logical(outp_v[sl], 1)

    def fire(t):
        sl = pl.ds(t * CHUNK, CHUNK)
        return (pltpu.async_copy(win_hbm.at[rin_v.at[sl]], win_b.at[t % 2], rsem),
                pltpu.async_copy(wout_hbm.at[rout_v.at[sl]], wout_b.at[t % 2], rsem))

    pend = fire(0)

    for k in range(NGRP):
        sl = pl.ds(k * L, L)
        a = inp_v[sl]
        b = outp_v[sl]
        lin_v[sl] = (lax.shift_left(lax.shift_right_logical(a, 3), 16) |
                     lax.shift_left(lax.shift_right_logical(b, 7), 10) |
                     lax.shift_left(lax.bitwise_and(a, 7), 7) |
                     lax.bitwise_and(b, 127))
        ob_v[sl] = b + N_CLASSES

    small = []
    for j in range(NCHUNK):
        sl = pl.ds(j * CHUNK, CHUNK)
        small.append(pltpu.async_copy(co_hbm.at[lin_v.at[sl]], co_v.at[sl], sem))
        small.append(pltpu.async_copy(bb_hbm.at[inp_v.at[sl]], bin_v.at[sl], sem))
        small.append(pltpu.async_copy(bb_hbm.at[ob_v.at[sl]], bout_v.at[sl], sem))

    for t in range(NPASS):
        nxt = fire(t + 1) if t + 1 < NPASS else None
        pend[0].wait()
        pend[1].wait()
        pend = nxt
        wbuf = win_b.at[t % 2]
        obuf = wout_b.at[t % 2]

        def dots(gl, _, wbuf=wbuf, obuf=obuf, t=t):
            sl = pl.ds(t * CHUNK + gl * L, L)
            ii = lax.iota(jnp.int32, L)
            rows = ii + gl * L
            cin = lax.bitwise_and(inp_v[sl], 1) * EMBED
            cout = lax.bitwise_and(outp_v[sl], 1) * EMBED
            acc4 = [jnp.zeros((L,), jnp.float32) for _ in range(4)]
            for d in range(EMBED):
                dd = lax.bitwise_and(ii + d, EMBED - 1)
                acc4[d % 4] = acc4[d % 4] + (
                    plsc.load_gather(wbuf, [rows, cin + dd]) *
                    plsc.load_gather(obuf, [rows, cout + dd]))
            pred_v[sl] = (acc4[0] + acc4[1]) + (acc4[2] + acc4[3])
            return 0

        lax.fori_loop(0, GPP, dots, 0, unroll=2)

    for c in small:
        c.wait()

    def loss(g, acc):
        sl = pl.ds(g * L, L)
        co = co_v[sl] + 1.0
        lnco = _vln(co)
        w = jnp.where(co > X_MAX, 1.0, jnp.exp(ALPHA * (lnco - LN_XMAX)))
        diff = pred_v[sl] + bin_v[sl] + bout_v[sl] - lnco
        return acc + diff * diff * w

    part_v[...] = lax.fori_loop(0, NGRP, loss, jnp.zeros((L,), jnp.float32),
                                unroll=2)
    pltpu.sync_copy(part_v, out_hbm.at[wid])


def kernel(input, output, co_oc, W_in, b_in, W_out, b_out):
    co_phys = co_oc.reshape(1024, 8, 64, 128).transpose(0, 2, 1, 3).reshape(-1)
    bb = jnp.concatenate([b_in.reshape(-1), b_out.reshape(-1)])
    parts = _glove_sc(input, output, co_phys,
                      W_in.reshape(N_CLASSES // 2, 2 * EMBED),
                      W_out.reshape(N_CLASSES // 2, 2 * EMBED), bb)
    return jnp.sum(parts)
